# baseline (speedup 1.0000x reference)
import functools

import jax
import jax.numpy as jnp
from jax import lax
from jax.experimental import pallas as pl
from jax.experimental.pallas import tpu as pltpu
from jax.experimental.pallas import tpu_sc as plsc

N_NODES = 100000
N_EDGES = 1600000
FDIM = 128
SDIM = 32
GDIM = 96
LDIM = 16
ROUNDS = 3

NGRP = 4
QROWS = N_NODES // NGRP
GRP = 500
GRID = QROWS // GRP
HALF_B = GRID // 2

NC = 2
NS = 16
CHUNK = 384

CHUNKS_PER_TILE = -(-N_EDGES // (CHUNK * 2 * NS)) * 2
CHUNKS_TOTAL = CHUNKS_PER_TILE * NS
E_PAD = CHUNKS_TOTAL * CHUNK

ACC_ROWS = 50176
STRIPE = ACC_ROWS // NS
HALF = 2 * QROWS
PAD_DST = 1 << 20


def _dot(a, b):
    return jnp.dot(a, b, preferred_element_type=jnp.float32)


def _tc_init_body(x0_ref, x1_ref, x2_ref, x3_ref, inW_ref, inb_ref, mW_ref,
                  mb_ref, whhT_ref, bhh_ref, state_ref, msg_ref, gh_ref):
    xs = [x0_ref, x1_ref, x2_ref, x3_ref]
    sts, ms, ghs = [], [], []
    for c in range(NGRP):
        st = jnp.maximum(_dot(xs[c][0], inW_ref[...]) + inb_ref[...], 0.0)
        sts.append(st)
        ms.append(jnp.maximum(_dot(st, mW_ref[...]) + mb_ref[...], 0.0))
        ghs.append(_dot(st, whhT_ref[...]) + bhh_ref[...])
    state_ref[0] = jnp.concatenate(sts, axis=1)
    msg_ref[0] = jnp.concatenate(ms, axis=1)
    gh_ref[0] = jnp.concatenate(ghs, axis=1)


def _read_agg(alo_ref, ahi_ref):
    pid = pl.program_id(0)
    return jnp.where(pid < HALF_B, alo_ref[0], ahi_ref[0])


def _gru_group(s_c, a_c, gh_c, w, bih):
    gi = _dot(a_c, w) + bih
    r = jax.nn.sigmoid(gi[:, :SDIM] + gh_c[:, :SDIM])
    z = jax.nn.sigmoid(gi[:, SDIM:2 * SDIM] + gh_c[:, SDIM:2 * SDIM])
    n = jnp.tanh(gi[:, 2 * SDIM:] + r * gh_c[:, 2 * SDIM:])
    return s_c + (1.0 - z) * n + z * s_c


def _tc_mid_body(state_ref, alo_ref, ahi_ref, gh_ref, wihT_ref, bih_ref,
                 mW_ref, mb_ref, whhT_ref, bhh_ref,
                 nstate_ref, msg_ref, ghn_ref):
    a = _read_agg(alo_ref, ahi_ref)
    w = wihT_ref[...]
    news, ms, ghs = [], [], []
    for c in range(NGRP):
        new = _gru_group(state_ref[0, :, SDIM * c:SDIM * (c + 1)],
                         a[:, SDIM * c:SDIM * (c + 1)],
                         gh_ref[0, :, GDIM * c:GDIM * (c + 1)], w, bih_ref[...])
        news.append(new)
        ms.append(jnp.maximum(_dot(new, mW_ref[...]) + mb_ref[...], 0.0))
        ghs.append(_dot(new, whhT_ref[...]) + bhh_ref[...])
    nstate_ref[0] = jnp.concatenate(news, axis=1)
    msg_ref[0] = jnp.concatenate(ms, axis=1)
    ghn_ref[0] = jnp.concatenate(ghs, axis=1)


def _tc_final_body(state_ref, alo_ref, ahi_ref, gh_ref, wihT_ref, bih_ref,
                   muW_ref, mub_ref, lsW_ref, lsb_ref, mu_ref, ls_ref):
    a = _read_agg(alo_ref, ahi_ref)
    w = wihT_ref[...]
    mus, lss = [], []
    for c in range(NGRP):
        new = _gru_group(state_ref[0, :, SDIM * c:SDIM * (c + 1)],
                         a[:, SDIM * c:SDIM * (c + 1)],
                         gh_ref[0, :, GDIM * c:GDIM * (c + 1)], w, bih_ref[...])
        mus.append(_dot(new, muW_ref[...]) + mub_ref[...])
        lss.append(_dot(new, lsW_ref[...]) + lsb_ref[...])
    mu_ref[0] = jnp.concatenate(mus, axis=1)
    ls_ref[0] = jnp.concatenate(lss, axis=1)


def _blk3(h, w):
    return pl.BlockSpec((1, h, w), lambda i: (i, 0, 0))


def _full_spec(shape):
    return pl.BlockSpec(shape, lambda i: (0,) * len(shape))


def _sds(*shape):
    return jax.ShapeDtypeStruct(shape, jnp.float32)


_STATE_B = _blk3(GRP, 128)
_GH_B = _blk3(GRP, NGRP * GDIM)
_ALO_B = pl.BlockSpec((1, GRP, 128),
                      lambda i: (jnp.minimum(i, HALF_B - 1), 0, 0))
_AHI_B = pl.BlockSpec((1, GRP, 128),
                      lambda i: (jnp.maximum(i - HALF_B, 0), 0, 0))
_PACK_OUTS = [_sds(GRID, GRP, 128), _sds(GRID, GRP, 128),
              _sds(GRID, GRP, NGRP * GDIM)]
_PACK_OUT_SPECS = [_STATE_B, _STATE_B, _GH_B]


def _xq_spec(c):
    return pl.BlockSpec((1, GRP, FDIM), lambda i, c=c: (i + GRID * c, 0, 0))


def _tc_init(x, inW, inb, mW, mb, whhT, bhh):
    return pl.pallas_call(
        _tc_init_body,
        grid=(GRID,),
        in_specs=[_xq_spec(0), _xq_spec(1), _xq_spec(2), _xq_spec(3),
                  _full_spec((FDIM, SDIM)), _full_spec((1, SDIM)),
                  _full_spec((SDIM, SDIM)), _full_spec((1, SDIM)),
                  _full_spec((SDIM, GDIM)), _full_spec((1, GDIM))],
        out_specs=_PACK_OUT_SPECS,
        out_shape=_PACK_OUTS,
    )(x, x, x, x, inW, inb, mW, mb, whhT, bhh)


def _tc_mid(state, alo, ahi, gh, wihT, bih, mW, mb, whhT, bhh):
    return pl.pallas_call(
        _tc_mid_body,
        grid=(GRID,),
        in_specs=[_STATE_B, _ALO_B, _AHI_B, _GH_B,
                  _full_spec((SDIM, GDIM)), _full_spec((1, GDIM)),
                  _full_spec((SDIM, SDIM)), _full_spec((1, SDIM)),
                  _full_spec((SDIM, GDIM)), _full_spec((1, GDIM))],
        out_specs=_PACK_OUT_SPECS,
        out_shape=_PACK_OUTS,
    )(state, alo, ahi, gh, wihT, bih, mW, mb, whhT, bhh)


def _tc_final(state, alo, ahi, gh, wihT, bih, muW, mub, lsW, lsb):
    return pl.pallas_call(
        _tc_final_body,
        grid=(GRID,),
        in_specs=[_STATE_B, _ALO_B, _AHI_B, _GH_B,
                  _full_spec((SDIM, GDIM)), _full_spec((1, GDIM)),
                  _full_spec((SDIM, LDIM)), _full_spec((1, LDIM)),
                  _full_spec((SDIM, LDIM)), _full_spec((1, LDIM))],
        out_specs=[_blk3(GRP, NGRP * LDIM), _blk3(GRP, NGRP * LDIM)],
        out_shape=[_sds(GRID, GRP, NGRP * LDIM), _sds(GRID, GRP, NGRP * LDIM)],
    )(state, alo, ahi, gh, wihT, bih, muW, mub, lsW, lsb)


def _sc_body(msg_hbm, idx_hbm, zero_hbm, alo_hbm, ahi_hbm,
             acc, iv0, iv1, ldv0, ldv1, rows0, rows1, gsem, ssem):
    c = lax.axis_index("c")
    s = lax.axis_index("s")

    pltpu.sync_copy(zero_hbm.at[pl.ds(s * STRIPE, STRIPE)],
                    acc.at[pl.ds(s * STRIPE, STRIPE)])
    plsc.subcore_barrier()

    msgv = msg_hbm
    accv = acc
    base = c * HALF
    cbase = s * CHUNKS_PER_TILE

    def _load_compute_fire(iv, ldv, chunk_i):
        pltpu.sync_copy(idx_hbm.at[chunk_i], iv)
        cp = pltpu.async_copy(msgv.at[iv.at[0]], rows0 if iv is iv0 else rows1,
                              gsem)
        del cp
        for i in range(CHUNK // 16):
            d = iv[1, pl.ds(16 * i, 16)] - base
            ok = (d >= 0) & (d < HALF)
            ldv[0, pl.ds(16 * i, 16)] = jnp.where(ok, d, HALF + (d & 127))

    def _drain_g_fire_s(iv, ldv, rows):
        pltpu.make_async_copy(msgv.at[iv.at[0]], rows, gsem).wait()
        pltpu.async_copy(rows, accv.at[ldv.at[0]], ssem, add=True)

    def _drain_s(ldv, rows):
        pltpu.make_async_copy(rows, accv.at[ldv.at[0]], ssem).wait()

    _load_compute_fire(iv0, ldv0, cbase)
    _load_compute_fire(iv1, ldv1, cbase + 1)

    def _pair(p, _):
        b0 = cbase + 2 * p
        _drain_g_fire_s(iv0, ldv0, rows0)
        _drain_g_fire_s(iv1, ldv1, rows1)
        _drain_s(ldv0, rows0)
        _load_compute_fire(iv0, ldv0, b0 + 2)
        _drain_s(ldv1, rows1)
        _load_compute_fire(iv1, ldv1, b0 + 3)
        return 0
    lax.fori_loop(0, CHUNKS_PER_TILE // 2 - 1, _pair, 0)
    _drain_g_fire_s(iv0, ldv0, rows0)
    _drain_g_fire_s(iv1, ldv1, rows1)
    _drain_s(ldv0, rows0)
    _drain_s(ldv1, rows1)

    plsc.subcore_barrier()

    @pl.when(c == 0)
    def _():
        pltpu.sync_copy(acc.at[pl.ds(s * STRIPE, STRIPE)],
                        alo_hbm.at[pl.ds(s * STRIPE, STRIPE)])

    @pl.when(c == 1)
    def _():
        pltpu.sync_copy(acc.at[pl.ds(s * STRIPE, STRIPE)],
                        ahi_hbm.at[pl.ds(s * STRIPE, STRIPE)])


@functools.cache
def _sc_aggregate_fn():
    return pl.kernel(
        _sc_body,
        out_type=[_sds(ACC_ROWS, SDIM), _sds(ACC_ROWS, SDIM)],
        mesh=plsc.VectorSubcoreMesh(core_axis_name="c", subcore_axis_name="s"),
        scratch_types=[
            pltpu.VMEM_SHARED((ACC_ROWS, SDIM), jnp.float32),
            pltpu.VMEM((2, CHUNK), jnp.int32),
            pltpu.VMEM((2, CHUNK), jnp.int32),
            pltpu.VMEM((1, CHUNK), jnp.int32),
            pltpu.VMEM((1, CHUNK), jnp.int32),
            pltpu.VMEM((CHUNK, SDIM), jnp.float32),
            pltpu.VMEM((CHUNK, SDIM), jnp.float32),
            pltpu.SemaphoreType.DMA,
            pltpu.SemaphoreType.DMA,
        ],
        compiler_params=pltpu.CompilerParams(use_tc_tiling_on_sc=False),
    )


def _sc_aggregate(msg_p, idx_comb, zeros):
    return _sc_aggregate_fn()(msg_p, idx_comb, zeros)


def kernel(x, edge_index, input_W, input_b, msg_W, msg_b, gru_wih, gru_whh,
           gru_bih, gru_bhh, mu_W, mu_b, ls_W, ls_b):
    pad = E_PAD - N_EDGES

    def _perm(n):
        q = n // QROWS
        return 4 * n - (4 * QROWS - 1) * q

    src = jnp.concatenate([_perm(edge_index[0]), jnp.zeros((pad,), jnp.int32)])
    dst = jnp.concatenate([_perm(edge_index[1]),
                           jnp.full((pad,), PAD_DST, jnp.int32)])
    idx_comb = jnp.stack([src.reshape(CHUNKS_TOTAL, CHUNK),
                          dst.reshape(CHUNKS_TOTAL, CHUNK)], axis=1)
    zeros = jnp.zeros((ACC_ROWS, SDIM), jnp.float32)

    inb = input_b.reshape(1, SDIM)
    mb = msg_b.reshape(ROUNDS, 1, SDIM)
    bih = gru_bih.reshape(ROUNDS, 1, GDIM)
    bhh = gru_bhh.reshape(ROUNDS, 1, GDIM)
    wihT = jnp.transpose(gru_wih, (0, 2, 1))
    whhT = jnp.transpose(gru_whh, (0, 2, 1))
    mub = mu_b.reshape(1, LDIM)
    lsb = ls_b.reshape(1, LDIM)

    x3 = x.reshape(GRID * NGRP, GRP, FDIM)
    state, msg, gh = _tc_init(x3, input_W, inb, msg_W[0], mb[0],
                              whhT[0], bhh[0])
    for r in range(ROUNDS):
        alo2, ahi2 = _sc_aggregate(msg.reshape(N_NODES, SDIM),
                                   idx_comb, zeros)
        alo = alo2[:HALF].reshape(HALF_B, GRP, 128)
        ahi = ahi2[:HALF].reshape(HALF_B, GRP, 128)
        if r < ROUNDS - 1:
            state, msg, gh = _tc_mid(state, alo, ahi, gh, wihT[r], bih[r],
                                     msg_W[r + 1], mb[r + 1],
                                     whhT[r + 1], bhh[r + 1])
        else:
            mu_p, ls_p = _tc_final(state, alo, ahi, gh, wihT[r], bih[r],
                                   mu_W, mub, ls_W, lsb)
    mu = mu_p.reshape(QROWS, NGRP, LDIM).transpose(1, 0, 2).reshape(N_NODES,
                                                                    LDIM)
    ls = ls_p.reshape(QROWS, NGRP, LDIM).transpose(1, 0, 2).reshape(N_NODES,
                                                                    LDIM)
    return (mu, ls)


# speedup vs baseline: 1.2051x; 1.2051x over previous
import functools

import jax
import jax.numpy as jnp
from jax import lax
from jax.experimental import pallas as pl
from jax.experimental.pallas import tpu as pltpu
from jax.experimental.pallas import tpu_sc as plsc

N_NODES = 100000
N_EDGES = 1600000
FDIM = 128
SDIM = 32
HDIM = 16
GDIM = 96
LDIM = 16
ROUNDS = 3

NGRP = 4
QROWS = N_NODES // NGRP
GRP = 500
GRID = QROWS // GRP

NC = 2
NS = 16
CHUNK = 512

CHUNKS_PER_TILE = -(-N_EDGES // (CHUNK * 2 * NS)) * 2
CHUNKS_TOTAL = CHUNKS_PER_TILE * NS
E_PAD = CHUNKS_TOTAL * CHUNK

ACC_ROWS = 100352
STRIPE = ACC_ROWS // NS
NTRASH = ACC_ROWS - N_NODES


def _dot(a, b):
    return jnp.dot(a, b, preferred_element_type=jnp.float32)


def _tc_init_body(x0_ref, x1_ref, x2_ref, x3_ref, inW_ref, inb_ref, mW_ref,
                  mb_ref, whhT_ref, bhh_ref, state_ref, msg_ref, gh_ref):
    xs = [x0_ref, x1_ref, x2_ref, x3_ref]
    sts, ms, ghs = [], [], []
    for c in range(NGRP):
        st = jnp.maximum(_dot(xs[c][0], inW_ref[...]) + inb_ref[...], 0.0)
        sts.append(st)
        ms.append(jnp.maximum(_dot(st, mW_ref[...]) + mb_ref[...], 0.0))
        ghs.append(_dot(st, whhT_ref[...]) + bhh_ref[...])
    state_ref[0] = jnp.concatenate(sts, axis=1)
    msg_ref[0] = jnp.concatenate(ms, axis=1)
    gh_ref[0] = jnp.concatenate(ghs, axis=1)


def _agg_group(alo_ref, ahi_ref, c):
    return jnp.concatenate([alo_ref[0, :, HDIM * c:HDIM * (c + 1)],
                            ahi_ref[0, :, HDIM * c:HDIM * (c + 1)]], axis=1)


def _gru_group(s_c, a_c, gh_c, w, bih):
    gi = _dot(a_c, w) + bih
    r = jax.nn.sigmoid(gi[:, :SDIM] + gh_c[:, :SDIM])
    z = jax.nn.sigmoid(gi[:, SDIM:2 * SDIM] + gh_c[:, SDIM:2 * SDIM])
    n = jnp.tanh(gi[:, 2 * SDIM:] + r * gh_c[:, 2 * SDIM:])
    return s_c + (1.0 - z) * n + z * s_c


def _tc_mid_body(state_ref, alo_ref, ahi_ref, gh_ref, wihT_ref, bih_ref,
                 mW_ref, mb_ref, whhT_ref, bhh_ref,
                 nstate_ref, msg_ref, ghn_ref):
    w = wihT_ref[...]
    news, ms, ghs = [], [], []
    for c in range(NGRP):
        new = _gru_group(state_ref[0, :, SDIM * c:SDIM * (c + 1)],
                         _agg_group(alo_ref, ahi_ref, c),
                         gh_ref[0, :, GDIM * c:GDIM * (c + 1)], w, bih_ref[...])
        news.append(new)
        ms.append(jnp.maximum(_dot(new, mW_ref[...]) + mb_ref[...], 0.0))
        ghs.append(_dot(new, whhT_ref[...]) + bhh_ref[...])
    nstate_ref[0] = jnp.concatenate(news, axis=1)
    msg_ref[0] = jnp.concatenate(ms, axis=1)
    ghn_ref[0] = jnp.concatenate(ghs, axis=1)


def _tc_final_body(state_ref, alo_ref, ahi_ref, gh_ref, wihT_ref, bih_ref,
                   muW_ref, mub_ref, lsW_ref, lsb_ref, mu_ref, ls_ref):
    w = wihT_ref[...]
    mus, lss = [], []
    for c in range(NGRP):
        new = _gru_group(state_ref[0, :, SDIM * c:SDIM * (c + 1)],
                         _agg_group(alo_ref, ahi_ref, c),
                         gh_ref[0, :, GDIM * c:GDIM * (c + 1)], w, bih_ref[...])
        mus.append(_dot(new, muW_ref[...]) + mub_ref[...])
        lss.append(_dot(new, lsW_ref[...]) + lsb_ref[...])
    mu_ref[0] = jnp.concatenate(mus, axis=1)
    ls_ref[0] = jnp.concatenate(lss, axis=1)


def _blk3(h, w):
    return pl.BlockSpec((1, h, w), lambda i: (i, 0, 0))


def _full_spec(shape):
    return pl.BlockSpec(shape, lambda i: (0,) * len(shape))


def _sds(*shape):
    return jax.ShapeDtypeStruct(shape, jnp.float32)


_STATE_B = _blk3(GRP, 128)
_GH_B = _blk3(GRP, NGRP * GDIM)
_AGG_B = _blk3(GRP, NGRP * HDIM)
_PACK_OUTS = [_sds(GRID, GRP, 128), _sds(GRID, GRP, 128),
              _sds(GRID, GRP, NGRP * GDIM)]
_PACK_OUT_SPECS = [_STATE_B, _STATE_B, _GH_B]


def _xq_spec(c):
    return pl.BlockSpec((1, GRP, FDIM), lambda i, c=c: (i + GRID * c, 0, 0))


def _tc_init(x, inW, inb, mW, mb, whhT, bhh):
    return pl.pallas_call(
        _tc_init_body,
        grid=(GRID,),
        in_specs=[_xq_spec(0), _xq_spec(1), _xq_spec(2), _xq_spec(3),
                  _full_spec((FDIM, SDIM)), _full_spec((1, SDIM)),
                  _full_spec((SDIM, SDIM)), _full_spec((1, SDIM)),
                  _full_spec((SDIM, GDIM)), _full_spec((1, GDIM))],
        out_specs=_PACK_OUT_SPECS,
        out_shape=_PACK_OUTS,
    )(x, x, x, x, inW, inb, mW, mb, whhT, bhh)


def _tc_mid(state, alo, ahi, gh, wihT, bih, mW, mb, whhT, bhh):
    return pl.pallas_call(
        _tc_mid_body,
        grid=(GRID,),
        in_specs=[_STATE_B, _AGG_B, _AGG_B, _GH_B,
                  _full_spec((SDIM, GDIM)), _full_spec((1, GDIM)),
                  _full_spec((SDIM, SDIM)), _full_spec((1, SDIM)),
                  _full_spec((SDIM, GDIM)), _full_spec((1, GDIM))],
        out_specs=_PACK_OUT_SPECS,
        out_shape=_PACK_OUTS,
    )(state, alo, ahi, gh, wihT, bih, mW, mb, whhT, bhh)


def _tc_final(state, alo, ahi, gh, wihT, bih, muW, mub, lsW, lsb):
    return pl.pallas_call(
        _tc_final_body,
        grid=(GRID,),
        in_specs=[_STATE_B, _AGG_B, _AGG_B, _GH_B,
                  _full_spec((SDIM, GDIM)), _full_spec((1, GDIM)),
                  _full_spec((SDIM, LDIM)), _full_spec((1, LDIM)),
                  _full_spec((SDIM, LDIM)), _full_spec((1, LDIM))],
        out_specs=[_blk3(GRP, NGRP * LDIM), _blk3(GRP, NGRP * LDIM)],
        out_shape=[_sds(GRID, GRP, NGRP * LDIM), _sds(GRID, GRP, NGRP * LDIM)],
    )(state, alo, ahi, gh, wihT, bih, muW, mub, lsW, lsb)


def _sc_body(msg_hbm, idx_hbm, zero_hbm, alo_hbm, ahi_hbm,
             acc, iv0, iv1, gv0, gv1, rows0, rows1, gsem, ssem):
    c = lax.axis_index("c")
    s = lax.axis_index("s")

    pltpu.sync_copy(zero_hbm.at[pl.ds(s * STRIPE, STRIPE)],
                    acc.at[pl.ds(s * STRIPE, STRIPE)])
    plsc.subcore_barrier()

    cbase = s * CHUNKS_PER_TILE

    def _load_compute_fire(iv, gv, rows, chunk_i):
        pltpu.sync_copy(idx_hbm.at[chunk_i], iv)
        for i in range(CHUNK // 16):
            gv[0, pl.ds(16 * i, 16)] = iv[0, pl.ds(16 * i, 16)] + c
        pltpu.async_copy(msg_hbm.at[gv.at[0]], rows, gsem)

    def _drain_g_fire_s(iv, gv, rows):
        pltpu.make_async_copy(msg_hbm.at[gv.at[0]], rows, gsem).wait()
        pltpu.async_copy(rows, acc.at[iv.at[1]], ssem, add=True)

    def _drain_s(iv, rows):
        pltpu.make_async_copy(rows, acc.at[iv.at[1]], ssem).wait()

    _load_compute_fire(iv0, gv0, rows0, cbase)
    _load_compute_fire(iv1, gv1, rows1, cbase + 1)

    def _pair(p, _):
        b0 = cbase + 2 * p
        _drain_g_fire_s(iv0, gv0, rows0)
        _drain_g_fire_s(iv1, gv1, rows1)
        _drain_s(iv0, rows0)
        _load_compute_fire(iv0, gv0, rows0, b0 + 2)
        _drain_s(iv1, rows1)
        _load_compute_fire(iv1, gv1, rows1, b0 + 3)
        return 0
    lax.fori_loop(0, CHUNKS_PER_TILE // 2 - 1, _pair, 0)
    _drain_g_fire_s(iv0, gv0, rows0)
    _drain_g_fire_s(iv1, gv1, rows1)
    _drain_s(iv0, rows0)
    _drain_s(iv1, rows1)

    plsc.subcore_barrier()

    @pl.when(c == 0)
    def _():
        pltpu.sync_copy(acc.at[pl.ds(s * STRIPE, STRIPE)],
                        alo_hbm.at[pl.ds(s * STRIPE, STRIPE)])

    @pl.when(c == 1)
    def _():
        pltpu.sync_copy(acc.at[pl.ds(s * STRIPE, STRIPE)],
                        ahi_hbm.at[pl.ds(s * STRIPE, STRIPE)])


@functools.cache
def _sc_aggregate_fn():
    return pl.kernel(
        _sc_body,
        out_type=[_sds(ACC_ROWS, HDIM), _sds(ACC_ROWS, HDIM)],
        mesh=plsc.VectorSubcoreMesh(core_axis_name="c", subcore_axis_name="s"),
        scratch_types=[
            pltpu.VMEM_SHARED((ACC_ROWS, HDIM), jnp.float32),
            pltpu.VMEM((2, CHUNK), jnp.int32),
            pltpu.VMEM((2, CHUNK), jnp.int32),
            pltpu.VMEM((1, CHUNK), jnp.int32),
            pltpu.VMEM((1, CHUNK), jnp.int32),
            pltpu.VMEM((CHUNK, HDIM), jnp.float32),
            pltpu.VMEM((CHUNK, HDIM), jnp.float32),
            pltpu.SemaphoreType.DMA,
            pltpu.SemaphoreType.DMA,
        ],
        compiler_params=pltpu.CompilerParams(use_tc_tiling_on_sc=False),
    )


def _sc_aggregate(msg, idx_comb, zeros):
    return _sc_aggregate_fn()(msg, idx_comb, zeros)


def kernel(x, edge_index, input_W, input_b, msg_W, msg_b, gru_wih, gru_whh,
           gru_bih, gru_bhh, mu_W, mu_b, ls_W, ls_b):
    pad = E_PAD - N_EDGES

    def _perm(n):
        q = ((n.astype(jnp.float32) + 0.5) *
             jnp.float32(1.0 / QROWS)).astype(jnp.int32)
        return 4 * n - (NGRP * QROWS - 1) * q

    src = jnp.concatenate([2 * _perm(edge_index[0]),
                           jnp.zeros((pad,), jnp.int32)])
    dst = jnp.concatenate([_perm(edge_index[1]),
                           N_NODES + jnp.arange(pad, dtype=jnp.int32)
                           % NTRASH])
    idx_comb = jnp.stack([src.reshape(CHUNKS_TOTAL, CHUNK),
                          dst.reshape(CHUNKS_TOTAL, CHUNK)], axis=1)
    zeros = jnp.zeros((ACC_ROWS, HDIM), jnp.float32)

    inb = input_b.reshape(1, SDIM)
    mb = msg_b.reshape(ROUNDS, 1, SDIM)
    bih = gru_bih.reshape(ROUNDS, 1, GDIM)
    bhh = gru_bhh.reshape(ROUNDS, 1, GDIM)
    wihT = jnp.transpose(gru_wih, (0, 2, 1))
    whhT = jnp.transpose(gru_whh, (0, 2, 1))
    mub = mu_b.reshape(1, LDIM)
    lsb = ls_b.reshape(1, LDIM)

    x3 = x.reshape(GRID * NGRP, GRP, FDIM)
    state, msg, gh = _tc_init(x3, input_W, inb, msg_W[0], mb[0],
                              whhT[0], bhh[0])
    for r in range(ROUNDS):
        alo2, ahi2 = _sc_aggregate(msg.reshape(2 * N_NODES, HDIM),
                                   idx_comb, zeros)
        alo = alo2[:N_NODES].reshape(GRID, GRP, NGRP * HDIM)
        ahi = ahi2[:N_NODES].reshape(GRID, GRP, NGRP * HDIM)
        if r < ROUNDS - 1:
            state, msg, gh = _tc_mid(state, alo, ahi, gh, wihT[r], bih[r],
                                     msg_W[r + 1], mb[r + 1],
                                     whhT[r + 1], bhh[r + 1])
        else:
            mu_p, ls_p = _tc_final(state, alo, ahi, gh, wihT[r], bih[r],
                                   mu_W, mub, ls_W, lsb)
    mu = mu_p.reshape(QROWS, NGRP, LDIM).transpose(1, 0, 2).reshape(N_NODES,
                                                                    LDIM)
    ls = ls_p.reshape(QROWS, NGRP, LDIM).transpose(1, 0, 2).reshape(N_NODES,
                                                                    LDIM)
    return (mu, ls)
